# baseline (speedup 1.0000x reference)
import functools

import jax
import jax.numpy as jnp
from jax import lax
from jax.experimental import pallas as pl
from jax.experimental.pallas import tpu as pltpu
from jax.experimental.pallas import tpu_sc as plsc

_K = 5
_IGNORE = -100
_ROWS = 128
_COLS = 100000
_LANES = 16
_NC = 2
_NS = 16
_NW = _NC * _NS
_ROWS_PER = _ROWS // _NW
_CHUNK = _COLS // 2
_NCHUNK = _ROWS_PER * 2
_UNROLL = 5
_NSTEP = _CHUNK // (_LANES * _UNROLL)


def _body(ypred_hbm, ytrue_hbm, hits_hbm, wsum_hbm,
          yt_v, buf0, buf1, v16, hit_v, w_v, sem0, sem1, vsem):
    wid = lax.axis_index("s") * _NC + lax.axis_index("c")
    bufs = (buf0, buf1)
    sems = (sem0, sem1)

    def chunk_copy(i):
        r, half = divmod(i, 2)
        row = wid * _ROWS_PER + r
        off = pl.multiple_of(row * _COLS + half * _CHUNK, 16)
        return pltpu.async_copy(
            ypred_hbm.at[pl.ds(off, _CHUNK)],
            bufs[i % 2], sems[i % 2])

    copies = [chunk_copy(0)]
    pltpu.sync_copy(ytrue_hbm, yt_v)

    zero_f = jnp.zeros((_LANES,), jnp.float32)
    zero_i = jnp.zeros((_LANES,), jnp.int32)
    false_m = jnp.zeros((_LANES,), jnp.bool_)
    lane_iota = lax.iota(jnp.int32, _LANES)

    hit_acc = zero_f
    w_acc = zero_f
    t_vec = zero_i
    v = zero_f
    acc = zero_i

    for i in range(_NCHUNK):
        r, half = divmod(i, 2)
        if i + 1 < _NCHUNK:
            copies.append(chunk_copy(i + 1))
        if half == 0:
            row = wid * _ROWS_PER + r
            row_vec = jnp.full((_LANES,), row, jnp.int32)
            t_vec = plsc.load_gather(yt_v, [row_vec])
            t_idx = jnp.clip(t_vec, 0, _COLS - 1)
            t16 = (t_idx // _LANES) * _LANES
            t16_s = jnp.max(t16)
            voff = pl.multiple_of(row * _COLS + t16_s, 16)
            pltpu.async_copy(ypred_hbm.at[pl.ds(voff, _LANES)],
                             v16, vsem).wait()
            v = plsc.load_gather(v16, [t_idx - t16])
            acc = zero_i
        copies[i].wait()
        buf = bufs[i % 2]

        def step(w, carry, _v=v, _buf=buf):
            a, tie = carry
            base = w * (_LANES * _UNROLL)
            for u in range(_UNROLL):
                x = _buf[pl.ds(base + u * _LANES, _LANES)]
                a = a + plsc.all_reduce_population_count(x > _v)
                tie = tie | (x == _v)
            return a, tie

        acc, tie = lax.fori_loop(0, _NSTEP, step, (acc, false_m))

        def fix(a, _v=v, _t=t_vec, _buf=buf, _base=half * _CHUNK):
            def cstep(w, aa):
                col = jnp.full((_LANES,), _base, jnp.int32) + w * _LANES + lane_iota
                x = _buf[pl.ds(w * _LANES, _LANES)]
                m = (x == _v) & (col < _t)
                return aa + plsc.all_reduce_population_count(m)
            return lax.fori_loop(0, _CHUNK // _LANES, cstep, a)

        has_tie = jnp.max(plsc.all_reduce_population_count(tie)) > 0
        acc = lax.cond(has_tie, fix, lambda a: a, acc)

        if half == 1:
            valid = t_vec != _IGNORE
            wf = jnp.where(valid, 1.0, 0.0).astype(jnp.float32)
            hit = jnp.where((acc < _K) & valid, 1.0, 0.0).astype(jnp.float32)
            hit_acc = hit_acc + hit
            w_acc = w_acc + wf

    hit_v[...] = hit_acc
    w_v[...] = w_acc
    pltpu.sync_copy(hit_v, hits_hbm.at[wid])
    pltpu.sync_copy(w_v, wsum_hbm.at[wid])


@jax.jit
def kernel(y_pred, y_true):
    mesh = plsc.VectorSubcoreMesh(core_axis_name="c", subcore_axis_name="s")
    f = functools.partial(
        pl.kernel,
        mesh=mesh,
        compiler_params=pltpu.CompilerParams(needs_layout_passes=False),
        out_type=[
            jax.ShapeDtypeStruct((_NW, _LANES), jnp.float32),
            jax.ShapeDtypeStruct((_NW, _LANES), jnp.float32),
        ],
        scratch_types=[
            pltpu.VMEM((_ROWS,), jnp.int32),
            pltpu.VMEM((_CHUNK,), jnp.float32),
            pltpu.VMEM((_CHUNK,), jnp.float32),
            pltpu.VMEM((_LANES,), jnp.float32),
            pltpu.VMEM((_LANES,), jnp.float32),
            pltpu.VMEM((_LANES,), jnp.float32),
            pltpu.SemaphoreType.DMA,
            pltpu.SemaphoreType.DMA,
            pltpu.SemaphoreType.DMA,
        ],
    )(_body)
    hits, ws = f(y_pred.reshape(-1), y_true.astype(jnp.int32))
    return (hits[:, 0].sum() / ws[:, 0].sum()) * 100.0


# speedup vs baseline: 1.4616x; 1.4616x over previous
import functools

import jax
import jax.numpy as jnp
from jax import lax
from jax.experimental import pallas as pl
from jax.experimental.pallas import tpu as pltpu
from jax.experimental.pallas import tpu_sc as plsc

_K = 5
_IGNORE = -100
_ROWS = 128
_COLS = 100000
_LANES = 16
_NC = 2
_NS = 16
_NW = _NC * _NS
_ROWS_PER = _ROWS // _NW
_CHUNK = 49920
_TAIL = _COLS - 2 * _CHUNK
_NCHUNK = _ROWS_PER * 2
_UNROLL = 5
_NSTEP = _CHUNK // (_LANES * _UNROLL)
_TSTEP = _TAIL // (_LANES * _UNROLL)


def _fast_count(buf, nstep, v, acc, tie):
    def step(w, carry):
        a, t = carry
        base = w * (_LANES * _UNROLL)
        for u in range(_UNROLL):
            x = buf[pl.ds(base + u * _LANES, _LANES)]
            a = a + plsc.all_reduce_population_count(x > v)
            t = t | (x == v)
        return a, t
    return lax.fori_loop(0, nstep, step, (acc, tie))


def _fix_count(buf, nvreg, col_base, v, t_vec, acc, lane_iota):
    def cstep(w, aa):
        col = jnp.full((_LANES,), col_base, jnp.int32) + w * _LANES + lane_iota
        x = buf[pl.ds(w * _LANES, _LANES)]
        m = (x == v) & (col < t_vec)
        return aa + plsc.all_reduce_population_count(m)
    return lax.fori_loop(0, nvreg, cstep, acc)


def _body(ypred_hbm, ytrue_hbm, hits_hbm, wsum_hbm,
          yt_v, buf0, buf1, tail_v, v128, hit_v, w_v,
          sem0, sem1, tsem, vsem):
    wid = lax.axis_index("s") * _NC + lax.axis_index("c")
    bufs = (buf0, buf1)
    sems = (sem0, sem1)

    def chunk_copy(i):
        r, half = divmod(i, 2)
        row = wid * _ROWS_PER + r
        return pltpu.async_copy(
            ypred_hbm.at[row].at[pl.ds(half * _CHUNK, _CHUNK)],
            bufs[i % 2], sems[i % 2])

    copies = [chunk_copy(0)]
    pltpu.sync_copy(ytrue_hbm, yt_v)

    zero_f = jnp.zeros((_LANES,), jnp.float32)
    zero_i = jnp.zeros((_LANES,), jnp.int32)
    false_m = jnp.zeros((_LANES,), jnp.bool_)
    lane_iota = lax.iota(jnp.int32, _LANES)

    hit_acc = zero_f
    w_acc = zero_f
    t_vec = zero_i
    t_idx = zero_i
    v = zero_f
    acc = zero_i

    for i in range(_NCHUNK):
        r, half = divmod(i, 2)
        row = wid * _ROWS_PER + r
        if i + 1 < _NCHUNK:
            copies.append(chunk_copy(i + 1))
        if half == 0:
            tail_cp = pltpu.async_copy(
                ypred_hbm.at[row].at[pl.ds(2 * _CHUNK, _TAIL)], tail_v, tsem)
            row_vec = jnp.full((_LANES,), row, jnp.int32)
            t_vec = plsc.load_gather(yt_v, [row_vec])
            t_idx = jnp.clip(t_vec, 0, _COLS - 1)
            t128 = jnp.minimum((t_idx // 128) * 128, 2 * _CHUNK - 128)
            t128_s = pl.multiple_of(jnp.max(t128), 128)
            pltpu.async_copy(
                ypred_hbm.at[row].at[pl.ds(t128_s, 128)], v128, vsem).wait()
            tail_cp.wait()
            v_lo = plsc.load_gather(v128, [t_idx - t128])
            v_hi = plsc.load_gather(
                tail_v, [jnp.clip(t_idx - 2 * _CHUNK, 0, _TAIL - 1)])
            v = jnp.where(t_idx < 2 * _CHUNK, v_lo, v_hi)
            acc = zero_i
        copies[i].wait()
        buf = bufs[i % 2]

        acc, tie = _fast_count(buf, _NSTEP, v, acc, false_m)
        has_tie = jnp.max(plsc.all_reduce_population_count(tie)) > 0
        acc = lax.cond(
            has_tie,
            functools.partial(_fix_count, buf, _CHUNK // _LANES,
                              half * _CHUNK, v, t_vec, lane_iota=lane_iota),
            lambda a: a, acc)

        if half == 1:
            acc, ttie = _fast_count(tail_v, _TSTEP, v, acc, false_m)
            has_ttie = jnp.max(plsc.all_reduce_population_count(ttie)) > 0
            acc = lax.cond(
                has_ttie,
                functools.partial(_fix_count, tail_v, _TAIL // _LANES,
                                  2 * _CHUNK, v, t_vec, lane_iota=lane_iota),
                lambda a: a, acc)
            valid = t_vec != _IGNORE
            wf = jnp.where(valid, 1.0, 0.0).astype(jnp.float32)
            hit = jnp.where((acc < _K) & valid, 1.0, 0.0).astype(jnp.float32)
            hit_acc = hit_acc + hit
            w_acc = w_acc + wf

    hit_v[...] = hit_acc
    w_v[...] = w_acc
    pltpu.sync_copy(hit_v, hits_hbm.at[wid])
    pltpu.sync_copy(w_v, wsum_hbm.at[wid])


@jax.jit
def kernel(y_pred, y_true):
    mesh = plsc.VectorSubcoreMesh(core_axis_name="c", subcore_axis_name="s")
    f = functools.partial(
        pl.kernel,
        mesh=mesh,
        compiler_params=pltpu.CompilerParams(needs_layout_passes=False),
        out_type=[
            jax.ShapeDtypeStruct((_NW, _LANES), jnp.float32),
            jax.ShapeDtypeStruct((_NW, _LANES), jnp.float32),
        ],
        scratch_types=[
            pltpu.VMEM((_ROWS,), jnp.int32),
            pltpu.VMEM((_CHUNK,), jnp.float32),
            pltpu.VMEM((_CHUNK,), jnp.float32),
            pltpu.VMEM((_TAIL,), jnp.float32),
            pltpu.VMEM((128,), jnp.float32),
            pltpu.VMEM((_LANES,), jnp.float32),
            pltpu.VMEM((_LANES,), jnp.float32),
            pltpu.SemaphoreType.DMA,
            pltpu.SemaphoreType.DMA,
            pltpu.SemaphoreType.DMA,
            pltpu.SemaphoreType.DMA,
        ],
    )(_body)
    hits, ws = f(y_pred, y_true.astype(jnp.int32))
    return (hits[:, 0].sum() / ws[:, 0].sum()) * 100.0
